# baseline (speedup 1.0000x reference)
import functools

import jax
import jax.numpy as jnp
from jax import lax
from jax.experimental import pallas as pl
from jax.experimental.pallas import tpu as pltpu
from jax.experimental.pallas import tpu_sc as plsc

D = 256
NT = 32
L = 16
GC = 64
SEC_ROWS = 16


def _make_sc_agg(n_sec, own, cap, out_rows):
    mesh = plsc.VectorSubcoreMesh(core_axis_name="c", subcore_axis_name="s")

    @functools.partial(
        pl.kernel,
        out_type=jax.ShapeDtypeStruct((out_rows, D), jnp.float32),
        mesh=mesh,
        scratch_types=[
            pltpu.VMEM((SEC_ROWS, 128), jnp.int32),
            pltpu.VMEM((SEC_ROWS, 128), jnp.int32),
            pltpu.VMEM((cap,), jnp.int32),
            pltpu.VMEM((cap,), jnp.int32),
            pltpu.VMEM((GC, D), jnp.float32),
            pltpu.VMEM((own + 8, D), jnp.float32),
            pltpu.SemaphoreType.DMA,
        ],
        compiler_params=pltpu.CompilerParams(needs_layout_passes=False),
    )
    def k(table, src2d, dst2d, out, sec_src, sec_dst, comp_src, comp_dst,
          buf, acc, sem):
        c = lax.axis_index("c")
        s = lax.axis_index("s")
        wid = c * 16 + s
        lo = wid * own
        iota = lax.iota(jnp.int32, L)
        zf = jnp.zeros((L,), jnp.float32)
        zi = jnp.zeros((L,), jnp.int32)
        trash = zi + own

        def zero_acc(i, carry):
            for t in range(D // L):
                acc[i, pl.ds(t * L, L)] = zf
            return carry

        lax.fori_loop(0, own + 1, zero_acc, None)

        def init_comp(j, carry):
            comp_src[pl.ds(j * L, L)] = zi
            comp_dst[pl.ds(j * L, L)] = trash
            return carry

        lax.fori_loop(0, cap // L, init_comp, None)

        def section(sec, cnt0):
            pltpu.sync_copy(src2d.at[pl.ds(sec * SEC_ROWS, SEC_ROWS)], sec_src)
            pltpu.sync_copy(dst2d.at[pl.ds(sec * SEC_ROWS, SEC_ROWS)], sec_dst)

            def row(j, cv):
                for t in range(128 // L):
                    col = t * L
                    d = sec_dst[j, pl.ds(col, L)]
                    sv = sec_src[j, pl.ds(col, L)]
                    dl = d - lo
                    m = (dl >= 0) & (dl < own)
                    pos = plsc.cumsum(jnp.where(m, 1, 0).astype(jnp.int32))
                    slot = jnp.minimum(cv + pos - 1, cap - 1)
                    plsc.store_scatter(comp_src, [slot], sv, mask=m)
                    plsc.store_scatter(comp_dst, [slot], dl, mask=m)
                    cv = cv + plsc.all_reduce_population_count(m)
                return cv

            return lax.fori_loop(0, SEC_ROWS, row, cnt0)

        lax.fori_loop(0, n_sec, section, jnp.zeros((L,), jnp.int32))

        one = jnp.full((L,), 1, jnp.int32)

        def chunk(kc, carry):
            pltpu.async_copy(
                table.at[comp_src.at[pl.ds(kc * GC, GC)]], buf, sem).wait()

            def row2(r2, carry2):
                dl = plsc.load_gather(comp_dst, [one * (kc * GC + r2)])
                for t in range(D // L):
                    v = buf[r2, pl.ds(t * L, L)]
                    plsc.addupdate_scatter(acc, [dl, iota + t * L], v)
                return carry2

            lax.fori_loop(0, GC, row2, None)
            return carry

        lax.fori_loop(0, cap // GC, chunk, None)

        pltpu.sync_copy(acc.at[pl.ds(0, own)], out.at[pl.ds(lo, own)])

    return k


def _mlp(xt, agg, W1, b1, W2, b2, m_pad, blk):
    def body(xt_ref, agg_ref, w1_ref, b1_ref, w2_ref, b2_ref, o_ref):
        h = xt_ref[...] + agg_ref[...]
        h = jnp.dot(h, w1_ref[...], preferred_element_type=jnp.float32)
        h = jnp.maximum(h + b1_ref[...], 0.0)
        h = jnp.dot(h, w2_ref[...], preferred_element_type=jnp.float32)
        o_ref[...] = jnp.maximum(h + b2_ref[...], 0.0)

    return pl.pallas_call(
        body,
        grid=(m_pad // blk,),
        in_specs=[
            pl.BlockSpec((blk, D), lambda i: (i, 0)),
            pl.BlockSpec((blk, D), lambda i: (i, 0)),
            pl.BlockSpec((D, D), lambda i: (0, 0)),
            pl.BlockSpec((1, D), lambda i: (0, 0)),
            pl.BlockSpec((D, D), lambda i: (0, 0)),
            pl.BlockSpec((1, D), lambda i: (0, 0)),
        ],
        out_specs=pl.BlockSpec((blk, D), lambda i: (i, 0)),
        out_shape=jax.ShapeDtypeStruct((m_pad, D), jnp.float32),
    )(xt, agg, W1, b1.reshape(1, D), W2, b2.reshape(1, D))


_TRASH = 2 ** 30


def _pad_edges(edge_index, e_real, e_pad):
    src = edge_index[0].astype(jnp.int32)
    dst = edge_index[1].astype(jnp.int32)
    src = jnp.pad(src, (0, e_pad - e_real))
    dst = jnp.pad(dst, (0, e_pad - e_real), constant_values=_TRASH)
    return src.reshape(-1, 128), dst.reshape(-1, 128)


def kernel(x, edge_index1, edge_index2,
           W1a, b1a, W2a, b2a, W1b, b1b, W2b, b2b):
    src1, dst1 = _pad_edges(edge_index1, 160000, 163840)
    sc1 = _make_sc_agg(n_sec=80, own=320, cap=5632, out_rows=10240)
    agg1 = sc1(x, src1, dst1)
    h1 = _mlp(x[:10240], agg1, W1a, b1a, W2a, b2a, 10240, 512)

    src2, dst2 = _pad_edges(edge_index2, 32000, 32768)
    sc2 = _make_sc_agg(n_sec=16, own=64, cap=1536, out_rows=2048)
    agg2 = sc2(h1, src2, dst2)
    h2 = _mlp(h1[:2048], agg2, W1b, b1b, W2b, b2b, 2048, 512)
    return h2[:2000]
